# baseline (speedup 1.0000x reference)
import functools

import jax
import jax.numpy as jnp
from jax import lax
from jax.experimental import pallas as pl
from jax.experimental.pallas import tpu as pltpu
from jax.experimental.pallas import tpu_sc as plsc

N = 10000
NPAD = 10112
CPAD = 10240
E = 320000
CHUNK = 128
NCH = 80
NTILES = 16
NCORES = 2
EPAD = NCORES * NTILES * NCH * CHUNK
RPT = NPAD // NTILES
CRPT = CPAD // NTILES

_f32 = jnp.float32
_MESH = plsc.VectorSubcoreMesh(core_axis_name="c", subcore_axis_name="s")


def _zero_vmem(ref, rows, d):
    for i in range(rows):
        for h in range(d // 16):
            ref[i, pl.ds(h * 16, 16)] = jnp.zeros((16,), _f32)


GCH = 32
GNCH = EPAD // (NCORES * NTILES * GCH)


def _make_sc_pass():
    d = 128
    scratch = [
        pltpu.VMEM((3 * GCH,), jnp.int32),
        pltpu.VMEM((GCH,), jnp.int32),
        pltpu.VMEM((1, GCH), jnp.int32),
        pltpu.VMEM((GCH, d), _f32),
        pltpu.VMEM((GCH, d), _f32),
        pltpu.VMEM((GCH, d), _f32),
        pltpu.VMEM((GCH + 16,), _f32),
        pltpu.VMEM((GCH + 16,), _f32),
        pltpu.VMEM((3 * d,), _f32),
        pltpu.VMEM((16,), jnp.int32),
        pltpu.VMEM((8, d), _f32),
        pltpu.VMEM((GCH,), _f32),
        pltpu.VMEM((CRPT,), _f32),
        pltpu.VMEM_SHARED((NPAD, d), _f32),
        pltpu.VMEM_SHARED((CPAD,), _f32),
        pltpu.SemaphoreType.DMA,
    ]

    def body(idx3_hbm, dst_hbm, ea0_hbm, ea1_hbm, t_hbm, we_hbm, flag_hbm,
             out_hbm, cnt_hbm, idx3, dstst, idx_d, buf0, buf1, buf2, ea0_v,
             ea1_v, we_v, flag_v, zbuf, ones_v, zc, acc, cacc, sem):
        c = lax.axis_index("c")
        s = lax.axis_index("s")
        wid = c * NTILES + s

        _zero_vmem(zbuf, 8, d)
        for h in range(GCH // 16):
            ones_v[pl.ds(h * 16, 16)] = jnp.ones((16,), _f32)
        for h in range(CRPT // 16):
            zc[pl.ds(h * 16, 16)] = jnp.zeros((16,), _f32)

        base = s * RPT

        def zero_acc(i, _):
            pltpu.sync_copy(zbuf, acc.at[pl.ds(base + i * 8, 8)])
            return ()
        lax.fori_loop(0, RPT // 8, zero_acc, ())
        pltpu.sync_copy(zc, cacc.at[pl.ds(s * CRPT, CRPT)])

        pltpu.sync_copy(we_hbm, we_v)
        pltpu.sync_copy(flag_hbm, flag_v)
        gated = flag_v[pl.ds(0, 16)][0] == 1
        plsc.subcore_barrier()

        def chunk(j, _):
            gbase = (wid * GNCH + j) * GCH
            pltpu.sync_copy(idx3_hbm.at[pl.ds(3 * gbase, 3 * GCH)], idx3)
            pltpu.sync_copy(dst_hbm.at[pl.ds(gbase, GCH)], dstst)
            for h in range(GCH // 16):
                idx_d[0, pl.ds(h * 16, 16)] = dstst[pl.ds(h * 16, 16)]
            pltpu.async_copy(t_hbm.at[idx3.at[pl.ds(0, GCH)]],
                             buf0, sem).wait()

            @pl.when(gated)
            def _():
                pltpu.async_copy(t_hbm.at[idx3.at[pl.ds(GCH, GCH)]],
                                 buf1, sem).wait()
                pltpu.async_copy(t_hbm.at[idx3.at[pl.ds(2 * GCH, GCH)]],
                                 buf2, sem).wait()
                pltpu.sync_copy(ea0_hbm.at[pl.ds(gbase, GCH)],
                                ea0_v.at[pl.ds(0, GCH)])
                pltpu.sync_copy(ea1_hbm.at[pl.ds(gbase, GCH)],
                                ea1_v.at[pl.ds(0, GCH)])

                def edge(i, _):
                    a0 = ea0_v[pl.ds(i, 16)][0]
                    a1 = ea1_v[pl.ds(i, 16)][0]
                    for h in range(d // 16):
                        sl = pl.ds(h * 16, 16)
                        t = (buf2[i, sl] + buf0[i, sl]
                             + a0 * we_v[pl.ds(h * 16, 16)]
                             + a1 * we_v[pl.ds(d + h * 16, 16)]
                             + we_v[pl.ds(2 * d + h * 16, 16)])
                        eta = 1.0 / (1.0 + jnp.exp(-t))
                        buf0[i, sl] = eta * buf1[i, sl]
                    return ()
                lax.fori_loop(0, GCH, edge, ())

            pltpu.sync_copy(buf0, acc.at[idx_d.at[0]], add=True)
            pltpu.sync_copy(ones_v, cacc.at[idx_d.at[0]], add=True)
            return ()
        lax.fori_loop(0, GNCH, chunk, ())

        plsc.subcore_barrier()

        def wb(i, _):
            pltpu.sync_copy(acc.at[pl.ds(base + i * 8, 8)], zbuf)
            pltpu.sync_copy(zbuf, out_hbm.at[pl.ds(c * NPAD + base + i * 8, 8)])
            return ()
        lax.fori_loop(0, RPT // 8, wb, ())
        pltpu.sync_copy(cacc.at[pl.ds(s * CRPT, CRPT)], zc)
        pltpu.sync_copy(zc, cnt_hbm.at[pl.ds(c * CPAD + s * CRPT, CRPT)])

    return pl.kernel(
        body,
        out_type=(jax.ShapeDtypeStruct((NCORES * NPAD, d), _f32),
                  jax.ShapeDtypeStruct((NCORES * CPAD,), _f32)),
        mesh=_MESH, scratch_types=scratch)


def _make_spmm_w8():
    D8 = 8
    ZB = 1264
    scratch = [
        pltpu.VMEM((D8, CHUNK), jnp.int32),
        pltpu.VMEM((D8, CHUNK), jnp.int32),
        pltpu.VMEM((D8, CHUNK), _f32),
        pltpu.VMEM((ZB,), _f32),
        pltpu.VMEM_SHARED((NPAD * D8,), _f32),
        pltpu.SemaphoreType.DMA,
    ]

    def body(srcw_hbm, dstw_hbm, x_hbm, out_hbm, idx_s, idx_d, rows, zbuf,
             acc, sem):
        c = lax.axis_index("c")
        s = lax.axis_index("s")
        wid = c * NTILES + s
        for h in range(ZB // 16):
            zbuf[pl.ds(h * 16, 16)] = jnp.zeros((16,), _f32)
        base = s * RPT * D8

        def za(i, _):
            pltpu.sync_copy(zbuf, acc.at[pl.ds(base + i * ZB, ZB)])
            return ()
        lax.fori_loop(0, 4, za, ())
        plsc.subcore_barrier()

        def chunk(j, _):
            pltpu.sync_copy(srcw_hbm.at[pl.ds((wid * NCH + j) * D8, D8)], idx_s)
            pltpu.sync_copy(dstw_hbm.at[pl.ds((wid * NCH + j) * D8, D8)], idx_d)
            descs = [pltpu.async_copy(x_hbm.at[idx_s.at[k]], rows.at[k], sem)
                     for k in range(D8)]
            for desc in descs:
                desc.wait()
            for k in range(D8):
                pltpu.sync_copy(rows.at[k], acc.at[idx_d.at[k]], add=True)
            return ()
        lax.fori_loop(0, NCH, chunk, ())

        plsc.subcore_barrier()

        def wbw(i, _):
            pltpu.sync_copy(acc.at[pl.ds(base + i * ZB, ZB)], zbuf)
            pltpu.sync_copy(
                zbuf, out_hbm.at[pl.ds(c * NPAD * D8 + base + i * ZB, ZB)])
            return ()
        lax.fori_loop(0, 4, wbw, ())

    return pl.kernel(body,
                     out_type=jax.ShapeDtypeStruct((NCORES * NPAD * 8,), _f32),
                     mesh=_MESH, scratch_types=scratch)


def _make_count():
    scratch = [
        pltpu.VMEM((1, CHUNK), jnp.int32),
        pltpu.VMEM((CHUNK,), jnp.int32),
        pltpu.VMEM((CHUNK,), _f32),
        pltpu.VMEM((CRPT,), _f32),
        pltpu.VMEM_SHARED((CPAD,), _f32),
    ]

    def body(dst_hbm, cnt_hbm, idx_d, dstst, ones_v, zc, cacc):
        c = lax.axis_index("c")
        s = lax.axis_index("s")
        wid = c * NTILES + s
        for h in range(CHUNK // 16):
            ones_v[pl.ds(h * 16, 16)] = jnp.ones((16,), _f32)
        for h in range(CRPT // 16):
            zc[pl.ds(h * 16, 16)] = jnp.zeros((16,), _f32)
        pltpu.sync_copy(zc, cacc.at[pl.ds(s * CRPT, CRPT)])
        plsc.subcore_barrier()

        def chunk(j, _):
            pltpu.sync_copy(dst_hbm.at[pl.ds((wid * NCH + j) * CHUNK, CHUNK)],
                            dstst)
            for h in range(CHUNK // 16):
                idx_d[0, pl.ds(h * 16, 16)] = dstst[pl.ds(h * 16, 16)]
            pltpu.sync_copy(ones_v, cacc.at[idx_d.at[0]], add=True)
            return ()
        lax.fori_loop(0, NCH, chunk, ())

        plsc.subcore_barrier()
        pltpu.sync_copy(cacc.at[pl.ds(s * CRPT, CRPT)], zc)
        pltpu.sync_copy(zc, cnt_hbm.at[pl.ds(c * CPAD + s * CRPT, CRPT)])

    return pl.kernel(body, out_type=jax.ShapeDtypeStruct((NCORES * CPAD,), _f32),
                     mesh=_MESH, scratch_types=scratch)


_sc_pass = _make_sc_pass()
_spmm_w8 = _make_spmm_w8()
_count = _make_count()


_BM = 400


@functools.partial(jax.jit, static_argnames=("relu", "nterms", "has_x0"))
def _tc_fused(xs, ws, b, x0, *, relu, nterms, has_x0):
    del nterms

    def body(*refs):
        n = len(xs)
        x_refs = refs[:n]
        w_refs = refs[n:2 * n]
        b_ref = refs[2 * n]
        pos = 2 * n + 1
        if has_x0:
            x0_ref = refs[pos]
            pos += 1
        out_ref = refs[pos]
        acc = jnp.zeros_like(out_ref)
        for xr, wr in zip(x_refs, w_refs):
            acc = acc + jnp.dot(xr[...], wr[...], preferred_element_type=_f32)
        acc = acc + b_ref[...]
        if has_x0:
            acc = acc + x0_ref[...]
        if relu:
            acc = jnp.maximum(acc, 0.0)
        out_ref[...] = acc

    in_specs = []
    for x in xs:
        k = x.shape[1]
        in_specs.append(pl.BlockSpec((_BM, k), lambda i: (i, 0)))
    for w in ws:
        k = w.shape[0]
        in_specs.append(pl.BlockSpec((k, 128), lambda i: (0, 0)))
    in_specs.append(pl.BlockSpec((1, 128), lambda i: (0, 0)))
    args = list(xs) + list(ws) + [b.reshape(1, 128)]
    if has_x0:
        in_specs.append(pl.BlockSpec((_BM, 128), lambda i: (i, 0)))
        args.append(x0)

    return pl.pallas_call(
        body,
        grid=(N // _BM,),
        in_specs=in_specs,
        out_specs=pl.BlockSpec((_BM, 128), lambda i: (i, 0)),
        out_shape=jax.ShapeDtypeStruct((N, 128), _f32),
    )(*args)


def _fused(xs_ws, b, x0=None, relu=True):
    xs = tuple(x for x, _ in xs_ws)
    ws = tuple(w for _, w in xs_ws)
    return _tc_fused(xs, ws, b, x0 if x0 is not None else jnp.zeros((1, 1), _f32),
                     relu=relu, nterms=len(xs), has_x0=x0 is not None)


def _pad_edges(row, col):
    ne = EPAD - row.shape[0]
    ar = jnp.arange(ne, dtype=col.dtype)
    row = jnp.concatenate([row, jnp.zeros((ne,), row.dtype)])
    col = jnp.concatenate([col, N + (ar % 64)])
    return row.astype(jnp.int32), col.astype(jnp.int32)


def _sum_parts(p, d):
    p = p.reshape(NCORES, -1, d) if d > 1 else p.reshape(NCORES, -1)
    return (p[0] + p[1])[:N]


def _r64(a):
    return a.reshape(-1, GCH)


def _r128(a):
    return a.reshape(-1, CHUNK)


def _idx3(srcflat):
    s = _r64(srcflat)
    return jnp.stack([s, s, s], axis=1).reshape(-1)


def _spmm(idx3, dst2d, t3, aux):
    zea, zwe, zflag = aux
    s, c = _sc_pass(idx3, dst2d, zea, zea, t3, zwe, zflag)
    return _sum_parts(s, 128), _sum_parts(c, 1)


def _dep(x, t):
    x, _ = lax.optimization_barrier((x, t[0, 0]))
    return x


def _expand_words(idx2d):
    w = idx2d[:, None, :] * 8 + jnp.arange(8, dtype=jnp.int32)[None, :, None]
    return w.reshape(-1, CHUNK)


def _spmm8(srcw, dstw, x):
    p = _spmm_w8(srcw, dstw, x.reshape(-1)).reshape(NCORES, NPAD, 8)
    return (p[0] + p[1])[:N]


def kernel(game_x, state_x, edge_index_v_v, edge_type_v_v, edge_index_history_v_s, edge_attr_history_v_s, edge_index_in_v_s, edge_index_s_s, W10, b10, Wrel1, Wroot1, b1, Wk3, bk3, Wq3, bq3, Wv3, bv3, We3, be3, Wskip3, b3, Wl32, bl32, Wr32, Wl4, bl4, Wr4, Wl42, bl42, Wr42, W2, b2, Wl5, bl5, Wr5, Wlin, blin):
    zea = jnp.zeros((EPAD,), _f32)
    zwe = jnp.zeros((3 * 128,), _f32)
    zflag = jnp.zeros((16,), jnp.int32)
    aux = (zea, zwe, zflag)
    zpad2n = jnp.zeros((2 * N, 128), _f32)
    vv_s, vv_d = _pad_edges(edge_index_v_v[0], edge_index_v_v[1])
    h_s, h_d = _pad_edges(edge_index_history_v_s[0], edge_index_history_v_s[1])
    in_s, in_d = _pad_edges(edge_index_in_v_s[0], edge_index_in_v_s[1])
    ss_s, ss_d = _pad_edges(edge_index_s_s[0], edge_index_s_s[1])

    zidx3 = jnp.zeros((3 * EPAD,), jnp.int32)
    ztab = jnp.zeros((3 * N, 128), _f32)
    _zs, deg = _sc_pass(zidx3, vv_d, zea, zea, ztab, zwe, zflag)
    deg = _sum_parts(deg, 1)
    dis = jnp.where(deg > 0, lax.rsqrt(jnp.maximum(deg, 1e-20)), 0.0)[:, None]
    vv_sw = _expand_words(_r128(vv_s))
    vv_dw = _expand_words(_r128(vv_d))
    x0 = jnp.pad(game_x, ((0, 0), (0, 1)))
    hs = [x0]
    h = x0
    for _ in range(3):
        s = _spmm8(vv_sw, vv_dw, dis * h)
        h = dis * s
        hs.append(h)
    X4 = jnp.concatenate(hs, axis=1)
    W10r = jnp.concatenate(
        [jnp.pad(W10[i], ((0, 1), (0, 0))) for i in range(4)], axis=0)
    gx = _fused([(X4, W10r)], b10, relu=True)

    et_pad = jnp.concatenate(
        [edge_type_v_v, jnp.full((EPAD - E,), -1, edge_type_v_v.dtype)])
    ar = (jnp.arange(EPAD, dtype=jnp.int32) % 64) + N
    dflat = vv_d
    vv_i3 = _idx3(vv_s)
    gx3 = jnp.concatenate([gx, zpad2n], axis=0)
    terms = []
    for r in range(3):
        colr = jnp.where(et_pad == r, dflat, ar)
        s, c = _spmm(vv_i3, colr, gx3, aux)
        terms.append((s / jnp.clip(c, 1.0)[:, None], Wrel1[r]))
    terms.append((gx, Wroot1))
    gx = _fused(terms, b1, relu=True)

    k_ = _fused([(jnp.pad(state_x, ((0, 0), (0, 1))),
                  jnp.pad(Wk3, ((0, 1), (0, 0))))], bk3, relu=False)
    q_ = _fused([(gx, Wq3)], bq3, relu=False)
    v_ = _fused([(gx, Wv3)], bv3, relu=False)
    h_dg = jnp.minimum(h_d, N - 1)
    ea_pad = jnp.concatenate(
        [edge_attr_history_v_s, jnp.zeros((EPAD - E, 2), _f32)])
    ea0 = ea_pad[:, 0]
    ea1 = ea_pad[:, 1]
    we = jnp.concatenate([We3[0], We3[1], be3])
    tqvk = jnp.concatenate([q_, v_, k_], axis=0)
    hr = _r64(h_s)
    src3 = jnp.stack([hr, N + hr, 2 * N + _r64(h_dg)], axis=1).reshape(-1)
    gflag = jnp.ones((16,), jnp.int32)
    p, _hc = _sc_pass(src3, h_d, ea0, ea1, tqvk, we, gflag)
    agg = _sum_parts(p, 128)
    sx = _fused([(jnp.pad(state_x, ((0, 0), (0, 1))),
                  jnp.pad(Wskip3, ((0, 1), (0, 0))))], b3, x0=agg, relu=True)

    gx3 = jnp.concatenate([gx, zpad2n], axis=0)
    h_i3 = _idx3(h_s)
    s, c = _spmm(h_i3, h_d, gx3, aux)
    m = s / jnp.clip(c, 1.0)[:, None]
    sx = _fused([(m, Wl32), (sx, Wr32)], bl32, relu=True)

    s, c = _spmm(_idx3(in_s), in_d, gx3, aux)
    m = s / jnp.clip(c, 1.0)[:, None]
    sx = _fused([(m, Wl4), (sx, Wr4)], bl4, relu=True)
    sx = _fused([(m, Wl42), (sx, Wr42)], bl42, relu=True)

    _zs, deg = _sc_pass(zidx3, ss_d, zea, zea, ztab, zwe, zflag)
    deg = _sum_parts(deg, 1)
    dis = jnp.where(deg > 0, lax.rsqrt(jnp.maximum(deg, 1e-20)), 0.0)[:, None]
    ss_i3 = _idx3(ss_s)
    hs = [sx]
    h = sx
    for _ in range(3):
        s, _ = _spmm(ss_i3, ss_d,
                     jnp.concatenate([dis * h, zpad2n], axis=0), aux)
        h = dis * s
        hs.append(h)
    sx = _fused([(hs[i], W2[i]) for i in range(4)], b2, relu=True)

    s, c = _spmm(ss_i3, ss_d, jnp.concatenate([sx, zpad2n], axis=0), aux)
    m = s / jnp.clip(c, 1.0)[:, None]
    sx = _fused([(m, Wl5), (sx, Wr5)], bl5, relu=True)

    Wlin_pad = jnp.pad(Wlin, ((0, 0), (0, 120)))
    blin_pad = jnp.pad(blin, (0, 120))
    out = _fused([(sx, Wlin_pad)], blin_pad, relu=False)
    return out[:, :8]


# speedup vs baseline: 2.3610x; 2.3610x over previous
import functools

import jax
import jax.numpy as jnp
from jax import lax
from jax.experimental import pallas as pl
from jax.experimental.pallas import tpu as pltpu
from jax.experimental.pallas import tpu_sc as plsc

N = 10000
NPAD = 10112
CPAD = 10240
E = 320000
CHUNK = 128
NCH = 80
NTILES = 16
NCORES = 2
EPAD = NCORES * NTILES * NCH * CHUNK
RPT = NPAD // NTILES
CRPT = CPAD // NTILES

_f32 = jnp.float32
_MESH = plsc.VectorSubcoreMesh(core_axis_name="c", subcore_axis_name="s")


def _zero_vmem(ref, rows, d):
    for i in range(rows):
        for h in range(d // 16):
            ref[i, pl.ds(h * 16, 16)] = jnp.zeros((16,), _f32)


GCH = 32
GNCH = EPAD // (NCORES * NTILES * GCH)


def _make_sc_pass():
    d = 128
    scratch = [
        pltpu.VMEM((3 * GCH,), jnp.int32),
        pltpu.VMEM((GCH,), jnp.int32),
        pltpu.VMEM((1, GCH), jnp.int32),
        pltpu.VMEM((GCH, d), _f32),
        pltpu.VMEM((GCH, d), _f32),
        pltpu.VMEM((GCH, d), _f32),
        pltpu.VMEM((GCH + 16,), _f32),
        pltpu.VMEM((GCH + 16,), _f32),
        pltpu.VMEM((3 * d,), _f32),
        pltpu.VMEM((16,), jnp.int32),
        pltpu.VMEM((8, d), _f32),
        pltpu.VMEM((GCH,), _f32),
        pltpu.VMEM((CRPT,), _f32),
        pltpu.VMEM_SHARED((NPAD, d), _f32),
        pltpu.VMEM_SHARED((CPAD,), _f32),
        pltpu.SemaphoreType.DMA,
    ]

    def body(idx3_hbm, dst_hbm, ea0_hbm, ea1_hbm, t_hbm, we_hbm, flag_hbm,
             out_hbm, cnt_hbm, idx3, dstst, idx_d, buf0, buf1, buf2, ea0_v,
             ea1_v, we_v, flag_v, zbuf, ones_v, zc, acc, cacc, sem):
        c = lax.axis_index("c")
        s = lax.axis_index("s")
        wid = c * NTILES + s

        _zero_vmem(zbuf, 8, d)
        for h in range(GCH // 16):
            ones_v[pl.ds(h * 16, 16)] = jnp.ones((16,), _f32)
        for h in range(CRPT // 16):
            zc[pl.ds(h * 16, 16)] = jnp.zeros((16,), _f32)

        base = s * RPT

        def zero_acc(i, _):
            pltpu.sync_copy(zbuf, acc.at[pl.ds(base + i * 8, 8)])
            return ()
        lax.fori_loop(0, RPT // 8, zero_acc, ())
        pltpu.sync_copy(zc, cacc.at[pl.ds(s * CRPT, CRPT)])

        pltpu.sync_copy(we_hbm, we_v)
        pltpu.sync_copy(flag_hbm, flag_v)
        gated = flag_v[pl.ds(0, 16)][0] == 1
        plsc.subcore_barrier()

        def chunk(j, _):
            gbase = (wid * GNCH + j) * GCH
            pltpu.sync_copy(idx3_hbm.at[pl.ds(3 * gbase, 3 * GCH)], idx3)
            pltpu.sync_copy(dst_hbm.at[pl.ds(gbase, GCH)], dstst)
            for h in range(GCH // 16):
                idx_d[0, pl.ds(h * 16, 16)] = dstst[pl.ds(h * 16, 16)]
            pltpu.async_copy(t_hbm.at[idx3.at[pl.ds(0, GCH)]],
                             buf0, sem).wait()

            @pl.when(gated)
            def _():
                pltpu.async_copy(t_hbm.at[idx3.at[pl.ds(GCH, GCH)]],
                                 buf1, sem).wait()
                pltpu.async_copy(t_hbm.at[idx3.at[pl.ds(2 * GCH, GCH)]],
                                 buf2, sem).wait()
                pltpu.sync_copy(ea0_hbm.at[pl.ds(gbase, GCH)],
                                ea0_v.at[pl.ds(0, GCH)])
                pltpu.sync_copy(ea1_hbm.at[pl.ds(gbase, GCH)],
                                ea1_v.at[pl.ds(0, GCH)])

                def edge(i, _):
                    a0 = ea0_v[pl.ds(i, 16)][0]
                    a1 = ea1_v[pl.ds(i, 16)][0]
                    for h in range(d // 16):
                        sl = pl.ds(h * 16, 16)
                        t = (buf2[i, sl] + buf0[i, sl]
                             + a0 * we_v[pl.ds(h * 16, 16)]
                             + a1 * we_v[pl.ds(d + h * 16, 16)]
                             + we_v[pl.ds(2 * d + h * 16, 16)])
                        eta = 1.0 / (1.0 + jnp.exp(-t))
                        buf0[i, sl] = eta * buf1[i, sl]
                    return ()
                lax.fori_loop(0, GCH, edge, ())

            pltpu.sync_copy(buf0, acc.at[idx_d.at[0]], add=True)
            pltpu.sync_copy(ones_v, cacc.at[idx_d.at[0]], add=True)
            return ()
        lax.fori_loop(0, GNCH, chunk, ())

        plsc.subcore_barrier()

        def wb(i, _):
            pltpu.sync_copy(acc.at[pl.ds(base + i * 8, 8)], zbuf)
            pltpu.sync_copy(zbuf, out_hbm.at[pl.ds(c * NPAD + base + i * 8, 8)])
            return ()
        lax.fori_loop(0, RPT // 8, wb, ())
        pltpu.sync_copy(cacc.at[pl.ds(s * CRPT, CRPT)], zc)
        pltpu.sync_copy(zc, cnt_hbm.at[pl.ds(c * CPAD + s * CRPT, CRPT)])

    return pl.kernel(
        body,
        out_type=(jax.ShapeDtypeStruct((NCORES * NPAD, d), _f32),
                  jax.ShapeDtypeStruct((NCORES * CPAD,), _f32)),
        mesh=_MESH, scratch_types=scratch)


def _make_spmm_w8():
    D8 = 8
    ZB = 1264
    scratch = [
        pltpu.VMEM((D8, CHUNK), jnp.int32),
        pltpu.VMEM((D8, CHUNK), jnp.int32),
        pltpu.VMEM((D8, CHUNK), _f32),
        pltpu.VMEM((ZB,), _f32),
        pltpu.VMEM_SHARED((NPAD * D8,), _f32),
        pltpu.SemaphoreType.DMA,
    ]

    def body(srcw_hbm, dstw_hbm, x_hbm, out_hbm, idx_s, idx_d, rows, zbuf,
             acc, sem):
        c = lax.axis_index("c")
        s = lax.axis_index("s")
        wid = c * NTILES + s
        for h in range(ZB // 16):
            zbuf[pl.ds(h * 16, 16)] = jnp.zeros((16,), _f32)
        base = s * RPT * D8

        def za(i, _):
            pltpu.sync_copy(zbuf, acc.at[pl.ds(base + i * ZB, ZB)])
            return ()
        lax.fori_loop(0, 4, za, ())
        plsc.subcore_barrier()

        def chunk(j, _):
            pltpu.sync_copy(srcw_hbm.at[pl.ds((wid * NCH + j) * D8, D8)], idx_s)
            pltpu.sync_copy(dstw_hbm.at[pl.ds((wid * NCH + j) * D8, D8)], idx_d)
            descs = [pltpu.async_copy(x_hbm.at[idx_s.at[k]], rows.at[k], sem)
                     for k in range(D8)]
            for desc in descs:
                desc.wait()
            for k in range(D8):
                pltpu.sync_copy(rows.at[k], acc.at[idx_d.at[k]], add=True)
            return ()
        lax.fori_loop(0, NCH, chunk, ())

        plsc.subcore_barrier()

        def wbw(i, _):
            pltpu.sync_copy(acc.at[pl.ds(base + i * ZB, ZB)], zbuf)
            pltpu.sync_copy(
                zbuf, out_hbm.at[pl.ds(c * NPAD * D8 + base + i * ZB, ZB)])
            return ()
        lax.fori_loop(0, 4, wbw, ())

    return pl.kernel(body,
                     out_type=jax.ShapeDtypeStruct((NCORES * NPAD * 8,), _f32),
                     mesh=_MESH, scratch_types=scratch)


def _make_count():
    scratch = [
        pltpu.VMEM((1, CHUNK), jnp.int32),
        pltpu.VMEM((CHUNK,), jnp.int32),
        pltpu.VMEM((CHUNK,), _f32),
        pltpu.VMEM((CRPT,), _f32),
        pltpu.VMEM_SHARED((CPAD,), _f32),
    ]

    def body(dst_hbm, cnt_hbm, idx_d, dstst, ones_v, zc, cacc):
        c = lax.axis_index("c")
        s = lax.axis_index("s")
        wid = c * NTILES + s
        for h in range(CHUNK // 16):
            ones_v[pl.ds(h * 16, 16)] = jnp.ones((16,), _f32)
        for h in range(CRPT // 16):
            zc[pl.ds(h * 16, 16)] = jnp.zeros((16,), _f32)
        pltpu.sync_copy(zc, cacc.at[pl.ds(s * CRPT, CRPT)])
        plsc.subcore_barrier()

        def chunk(j, _):
            pltpu.sync_copy(dst_hbm.at[pl.ds((wid * NCH + j) * CHUNK, CHUNK)],
                            dstst)
            for h in range(CHUNK // 16):
                idx_d[0, pl.ds(h * 16, 16)] = dstst[pl.ds(h * 16, 16)]
            pltpu.sync_copy(ones_v, cacc.at[idx_d.at[0]], add=True)
            return ()
        lax.fori_loop(0, NCH, chunk, ())

        plsc.subcore_barrier()
        pltpu.sync_copy(cacc.at[pl.ds(s * CRPT, CRPT)], zc)
        pltpu.sync_copy(zc, cnt_hbm.at[pl.ds(c * CPAD + s * CRPT, CRPT)])

    return pl.kernel(body, out_type=jax.ShapeDtypeStruct((NCORES * CPAD,), _f32),
                     mesh=_MESH, scratch_types=scratch)


_sc_pass = _make_sc_pass()
_spmm_w8 = _make_spmm_w8()
_count = _make_count()


_BM = 400


@functools.partial(jax.jit, static_argnames=("relu", "nterms", "has_x0"))
def _tc_fused(xs, ws, b, x0, *, relu, nterms, has_x0):
    del nterms

    def body(*refs):
        n = len(xs)
        x_refs = refs[:n]
        w_refs = refs[n:2 * n]
        b_ref = refs[2 * n]
        pos = 2 * n + 1
        if has_x0:
            x0_ref = refs[pos]
            pos += 1
        out_ref = refs[pos]
        acc = jnp.zeros_like(out_ref)
        for xr, wr in zip(x_refs, w_refs):
            acc = acc + jnp.dot(xr[...], wr[...], preferred_element_type=_f32)
        acc = acc + b_ref[...]
        if has_x0:
            acc = acc + x0_ref[...]
        if relu:
            acc = jnp.maximum(acc, 0.0)
        out_ref[...] = acc

    in_specs = []
    for x in xs:
        k = x.shape[1]
        in_specs.append(pl.BlockSpec((_BM, k), lambda i: (i, 0)))
    for w in ws:
        k = w.shape[0]
        in_specs.append(pl.BlockSpec((k, 128), lambda i: (0, 0)))
    in_specs.append(pl.BlockSpec((1, 128), lambda i: (0, 0)))
    args = list(xs) + list(ws) + [b.reshape(1, 128)]
    if has_x0:
        in_specs.append(pl.BlockSpec((_BM, 128), lambda i: (i, 0)))
        args.append(x0)

    return pl.pallas_call(
        body,
        grid=(N // _BM,),
        in_specs=in_specs,
        out_specs=pl.BlockSpec((_BM, 128), lambda i: (i, 0)),
        out_shape=jax.ShapeDtypeStruct((N, 128), _f32),
    )(*args)


def _fused(xs_ws, b, x0=None, relu=True):
    xs = tuple(x for x, _ in xs_ws)
    ws = tuple(w for _, w in xs_ws)
    return _tc_fused(xs, ws, b, x0 if x0 is not None else jnp.zeros((1, 1), _f32),
                     relu=relu, nterms=len(xs), has_x0=x0 is not None)


def _pad_edges(row, col):
    ne = EPAD - row.shape[0]
    ar = jnp.arange(ne, dtype=col.dtype)
    row = jnp.concatenate([row, jnp.zeros((ne,), row.dtype)])
    col = jnp.concatenate([col, N + (ar % 64)])
    return row.astype(jnp.int32), col.astype(jnp.int32)


def _sum_parts(p, d):
    p = p.reshape(NCORES, -1, d) if d > 1 else p.reshape(NCORES, -1)
    return (p[0] + p[1])[:N]


def _r64(a):
    return a.reshape(-1, GCH)


def _r128(a):
    return a.reshape(-1, CHUNK)


def _idx3(srcflat):
    s = _r64(srcflat)
    return jnp.stack([s, s, s], axis=1).reshape(-1)


def _spmm(idx3, dst2d, t3, aux):
    zea, zwe, zflag = aux
    s, c = _sc_pass(idx3, dst2d, zea, zea, t3, zwe, zflag)
    return _sum_parts(s, 128), _sum_parts(c, 1)


def _dep(x, t):
    x, _ = lax.optimization_barrier((x, t[0, 0]))
    return x


def _expand_words(idx2d):
    w = idx2d[:, None, :] * 8 + jnp.arange(8, dtype=jnp.int32)[None, :, None]
    return w.reshape(-1, CHUNK)


def _spmm8(srcw, dstw, x):
    p = _spmm_w8(srcw, dstw, x.reshape(-1)).reshape(NCORES, NPAD, 8)
    return (p[0] + p[1])[:N]


def kernel(game_x, state_x, edge_index_v_v, edge_type_v_v, edge_index_history_v_s, edge_attr_history_v_s, edge_index_in_v_s, edge_index_s_s, W10, b10, Wrel1, Wroot1, b1, Wk3, bk3, Wq3, bq3, Wv3, bv3, We3, be3, Wskip3, b3, Wl32, bl32, Wr32, Wl4, bl4, Wr4, Wl42, bl42, Wr42, W2, b2, Wl5, bl5, Wr5, Wlin, blin):
    zea = jnp.zeros((EPAD,), _f32)
    zwe = jnp.zeros((3 * 128,), _f32)
    zflag = jnp.zeros((16,), jnp.int32)
    aux = (zea, zwe, zflag)
    zpad2n = jnp.zeros((2 * N, 128), _f32)
    vv_s, vv_d = _pad_edges(edge_index_v_v[0], edge_index_v_v[1])
    h_s, h_d = _pad_edges(edge_index_history_v_s[0], edge_index_history_v_s[1])
    in_s, in_d = _pad_edges(edge_index_in_v_s[0], edge_index_in_v_s[1])
    ss_s, ss_d = _pad_edges(edge_index_s_s[0], edge_index_s_s[1])

    ztab = jnp.zeros((3 * N, 128), _f32)
    vv_i3 = _idx3(vv_s)
    _zs, deg = _sc_pass(vv_i3, vv_d, zea, zea, ztab, zwe, zflag)
    deg = _sum_parts(deg, 1)
    dis = jnp.where(deg > 0, lax.rsqrt(jnp.maximum(deg, 1e-20)), 0.0)[:, None]
    vv_sw = _expand_words(_r128(vv_s))
    vv_dw = _expand_words(_r128(vv_d))
    x0 = jnp.pad(game_x, ((0, 0), (0, 1)))
    hs = [x0]
    h = x0
    for _ in range(3):
        s = _spmm8(vv_sw, vv_dw, dis * h)
        h = dis * s
        hs.append(h)
    X4 = jnp.concatenate(hs, axis=1)
    W10r = jnp.concatenate(
        [jnp.pad(W10[i], ((0, 1), (0, 0))) for i in range(4)], axis=0)
    gx = _fused([(X4, W10r)], b10, relu=True)

    et_pad = jnp.concatenate(
        [edge_type_v_v, jnp.full((EPAD - E,), -1, edge_type_v_v.dtype)])
    ar = (jnp.arange(EPAD, dtype=jnp.int32) % 64) + N
    dflat = vv_d
    gx3 = jnp.concatenate([gx, zpad2n], axis=0)
    terms = []
    for r in range(3):
        colr = jnp.where(et_pad == r, dflat, ar)
        s, c = _spmm(vv_i3, colr, gx3, aux)
        terms.append((s / jnp.clip(c, 1.0)[:, None], Wrel1[r]))
    terms.append((gx, Wroot1))
    gx = _fused(terms, b1, relu=True)

    k_ = _fused([(jnp.pad(state_x, ((0, 0), (0, 1))),
                  jnp.pad(Wk3, ((0, 1), (0, 0))))], bk3, relu=False)
    q_ = _fused([(gx, Wq3)], bq3, relu=False)
    v_ = _fused([(gx, Wv3)], bv3, relu=False)
    h_dg = jnp.minimum(h_d, N - 1)
    ea_pad = jnp.concatenate(
        [edge_attr_history_v_s, jnp.zeros((EPAD - E, 2), _f32)])
    ea0 = ea_pad[:, 0]
    ea1 = ea_pad[:, 1]
    we = jnp.concatenate([We3[0], We3[1], be3])
    tqvk = jnp.concatenate([q_, v_, k_], axis=0)
    hr = _r64(h_s)
    src3 = jnp.stack([hr, N + hr, 2 * N + _r64(h_dg)], axis=1).reshape(-1)
    gflag = jnp.ones((16,), jnp.int32)
    p, _hc = _sc_pass(src3, h_d, ea0, ea1, tqvk, we, gflag)
    agg = _sum_parts(p, 128)
    sx = _fused([(jnp.pad(state_x, ((0, 0), (0, 1))),
                  jnp.pad(Wskip3, ((0, 1), (0, 0))))], b3, x0=agg, relu=True)

    gx3 = jnp.concatenate([gx, zpad2n], axis=0)
    h_i3 = _idx3(h_s)
    s, c = _spmm(h_i3, h_d, gx3, aux)
    m = s / jnp.clip(c, 1.0)[:, None]
    sx = _fused([(m, Wl32), (sx, Wr32)], bl32, relu=True)

    s, c = _spmm(_idx3(in_s), in_d, gx3, aux)
    m = s / jnp.clip(c, 1.0)[:, None]
    sx = _fused([(m, Wl4), (sx, Wr4)], bl4, relu=True)
    sx = _fused([(m, Wl42), (sx, Wr42)], bl42, relu=True)

    ss_i3 = _idx3(ss_s)
    _zs, deg = _sc_pass(ss_i3, ss_d, zea, zea, ztab, zwe, zflag)
    deg = _sum_parts(deg, 1)
    dis = jnp.where(deg > 0, lax.rsqrt(jnp.maximum(deg, 1e-20)), 0.0)[:, None]
    hs = [sx]
    h = sx
    for _ in range(3):
        s, _ = _spmm(ss_i3, ss_d,
                     jnp.concatenate([dis * h, zpad2n], axis=0), aux)
        h = dis * s
        hs.append(h)
    sx = _fused([(hs[i], W2[i]) for i in range(4)], b2, relu=True)

    s, c = _spmm(ss_i3, ss_d, jnp.concatenate([sx, zpad2n], axis=0), aux)
    m = s / jnp.clip(c, 1.0)[:, None]
    sx = _fused([(m, Wl5), (sx, Wr5)], bl5, relu=True)

    Wlin_pad = jnp.pad(Wlin, ((0, 0), (0, 120)))
    blin_pad = jnp.pad(blin, (0, 120))
    out = _fused([(sx, Wlin_pad)], blin_pad, relu=False)
    return out[:, :8]


# speedup vs baseline: 3.3968x; 1.4387x over previous
import functools

import jax
import jax.numpy as jnp
from jax import lax
from jax.experimental import pallas as pl
from jax.experimental.pallas import tpu as pltpu
from jax.experimental.pallas import tpu_sc as plsc

N = 10000
NPAD = 10112
CPAD = 10240
E = 320000
CHUNK = 128
NCH = 80
NTILES = 16
NCORES = 2
EPAD = NCORES * NTILES * NCH * CHUNK
RPT = NPAD // NTILES
CRPT = CPAD // NTILES

_f32 = jnp.float32
_MESH = plsc.VectorSubcoreMesh(core_axis_name="c", subcore_axis_name="s")


def _zero_vmem(ref, rows, d):
    for i in range(rows):
        for h in range(d // 16):
            ref[i, pl.ds(h * 16, 16)] = jnp.zeros((16,), _f32)


GCH = 32
GNCH = EPAD // (NCORES * NTILES * GCH)


def _make_sc_pass():
    d = 128
    scratch = [
        pltpu.VMEM((8 * 3 * GCH,), jnp.int32),
        pltpu.VMEM((8 * GCH,), jnp.int32),
        pltpu.VMEM((1, GCH), jnp.int32),
        pltpu.VMEM((GCH, d), _f32),
        pltpu.VMEM((GCH, d), _f32),
        pltpu.VMEM((GCH, d), _f32),
        pltpu.VMEM((GCH, d), _f32),
        pltpu.VMEM((GCH + 16,), _f32),
        pltpu.VMEM((GCH + 16,), _f32),
        pltpu.VMEM((3 * d,), _f32),
        pltpu.VMEM((16,), jnp.int32),
        pltpu.VMEM((8, d), _f32),
        pltpu.VMEM((GCH,), _f32),
        pltpu.VMEM((CRPT,), _f32),
        pltpu.VMEM_SHARED((NPAD, d), _f32),
        pltpu.VMEM_SHARED((CPAD,), _f32),
        pltpu.SemaphoreType.DMA,
    ]

    def body(idx3_hbm, dst_hbm, ea0_hbm, ea1_hbm, t_hbm, we_hbm, flag_hbm,
             out_hbm, cnt_hbm, idx3, dstst, idx_d, buf0a, buf0b, buf1, buf2,
             ea0_v, ea1_v, we_v, flag_v, zbuf, ones_v, zc, acc, cacc, sem):
        c = lax.axis_index("c")
        s = lax.axis_index("s")
        wid = c * NTILES + s

        _zero_vmem(zbuf, 8, d)
        for h in range(GCH // 16):
            ones_v[pl.ds(h * 16, 16)] = jnp.ones((16,), _f32)
        for h in range(CRPT // 16):
            zc[pl.ds(h * 16, 16)] = jnp.zeros((16,), _f32)

        base = s * RPT

        def zero_acc(i, _):
            pltpu.sync_copy(zbuf, acc.at[pl.ds(base + i * 8, 8)])
            return ()
        lax.fori_loop(0, RPT // 8, zero_acc, ())
        pltpu.sync_copy(zc, cacc.at[pl.ds(s * CRPT, CRPT)])

        pltpu.sync_copy(we_hbm, we_v)
        pltpu.sync_copy(flag_hbm, flag_v)
        gated = flag_v[pl.ds(0, 16)][0] == 1
        plsc.subcore_barrier()

        def load_blk(bj):
            pltpu.sync_copy(
                idx3_hbm.at[pl.ds((wid * GNCH + bj) * 3 * GCH, 8 * 3 * GCH)],
                idx3)
            pltpu.sync_copy(
                dst_hbm.at[pl.ds((wid * GNCH + bj) * GCH, 8 * GCH)], dstst)

        def fire(j, buf):
            return pltpu.async_copy(
                t_hbm.at[idx3.at[pl.ds((j % 8) * 3 * GCH, GCH)]], buf, sem)

        load_blk(0)
        fire(0, buf0a)

        def process(j, cur, nxt):
            boff = (j % 8) * GCH
            for h in range(GCH // 16):
                idx_d[0, pl.ds(h * 16, 16)] = dstst[pl.ds(boff + h * 16, 16)]
            pltpu.make_async_copy(t_hbm.at[pl.ds(0, GCH)], cur, sem).wait()

            @pl.when(gated)
            def _():
                pltpu.async_copy(
                    t_hbm.at[idx3.at[pl.ds((j % 8) * 3 * GCH + GCH, GCH)]],
                    buf1, sem).wait()
                pltpu.async_copy(
                    t_hbm.at[idx3.at[pl.ds((j % 8) * 3 * GCH + 2 * GCH, GCH)]],
                    buf2, sem).wait()
                gbase = (wid * GNCH + j) * GCH
                pltpu.sync_copy(ea0_hbm.at[pl.ds(gbase, GCH)],
                                ea0_v.at[pl.ds(0, GCH)])
                pltpu.sync_copy(ea1_hbm.at[pl.ds(gbase, GCH)],
                                ea1_v.at[pl.ds(0, GCH)])

                def edge(i, _):
                    a0 = ea0_v[pl.ds(i, 16)][0]
                    a1 = ea1_v[pl.ds(i, 16)][0]
                    for h in range(d // 16):
                        sl = pl.ds(h * 16, 16)
                        t = (buf2[i, sl] + cur[i, sl]
                             + a0 * we_v[pl.ds(h * 16, 16)]
                             + a1 * we_v[pl.ds(d + h * 16, 16)]
                             + we_v[pl.ds(2 * d + h * 16, 16)])
                        eta = 1.0 / (1.0 + jnp.exp(-t))
                        cur[i, sl] = eta * buf1[i, sl]
                    return ()
                lax.fori_loop(0, GCH, edge, ())

            @pl.when(j + 1 < GNCH)
            def _():
                @pl.when((j + 1) % 8 == 0)
                def _():
                    load_blk(j + 1)
                fire(j + 1, nxt)

            pltpu.sync_copy(cur, acc.at[idx_d.at[0]], add=True)
            pltpu.sync_copy(ones_v, cacc.at[idx_d.at[0]], add=True)

        def pair(i, _):
            process(2 * i, buf0a, buf0b)
            process(2 * i + 1, buf0b, buf0a)
            return ()
        lax.fori_loop(0, GNCH // 2, pair, ())

        plsc.subcore_barrier()

        def wb(i, _):
            pltpu.sync_copy(acc.at[pl.ds(base + i * 8, 8)], zbuf)
            pltpu.sync_copy(zbuf, out_hbm.at[pl.ds(c * NPAD + base + i * 8, 8)])
            return ()
        lax.fori_loop(0, RPT // 8, wb, ())
        pltpu.sync_copy(cacc.at[pl.ds(s * CRPT, CRPT)], zc)
        pltpu.sync_copy(zc, cnt_hbm.at[pl.ds(c * CPAD + s * CRPT, CRPT)])

    return pl.kernel(
        body,
        out_type=(jax.ShapeDtypeStruct((NCORES * NPAD, d), _f32),
                  jax.ShapeDtypeStruct((NCORES * CPAD,), _f32)),
        mesh=_MESH, scratch_types=scratch)


def _make_spmm_w8():
    D8 = 8
    ZB = 1264
    scratch = [
        pltpu.VMEM((D8, CHUNK), jnp.int32),
        pltpu.VMEM((D8, CHUNK), jnp.int32),
        pltpu.VMEM((D8, CHUNK), _f32),
        pltpu.VMEM((ZB,), _f32),
        pltpu.VMEM_SHARED((NPAD * D8,), _f32),
        pltpu.SemaphoreType.DMA,
    ]

    def body(srcw_hbm, dstw_hbm, x_hbm, out_hbm, idx_s, idx_d, rows, zbuf,
             acc, sem):
        c = lax.axis_index("c")
        s = lax.axis_index("s")
        wid = c * NTILES + s
        for h in range(ZB // 16):
            zbuf[pl.ds(h * 16, 16)] = jnp.zeros((16,), _f32)
        base = s * RPT * D8

        def za(i, _):
            pltpu.sync_copy(zbuf, acc.at[pl.ds(base + i * ZB, ZB)])
            return ()
        lax.fori_loop(0, 4, za, ())
        plsc.subcore_barrier()

        def chunk(j, _):
            pltpu.sync_copy(srcw_hbm.at[pl.ds((wid * NCH + j) * D8, D8)], idx_s)
            pltpu.sync_copy(dstw_hbm.at[pl.ds((wid * NCH + j) * D8, D8)], idx_d)
            descs = [pltpu.async_copy(x_hbm.at[idx_s.at[k]], rows.at[k], sem)
                     for k in range(D8)]
            for desc in descs:
                desc.wait()
            for k in range(D8):
                pltpu.sync_copy(rows.at[k], acc.at[idx_d.at[k]], add=True)
            return ()
        lax.fori_loop(0, NCH, chunk, ())

        plsc.subcore_barrier()

        def wbw(i, _):
            pltpu.sync_copy(acc.at[pl.ds(base + i * ZB, ZB)], zbuf)
            pltpu.sync_copy(
                zbuf, out_hbm.at[pl.ds(c * NPAD * D8 + base + i * ZB, ZB)])
            return ()
        lax.fori_loop(0, 4, wbw, ())

    return pl.kernel(body,
                     out_type=jax.ShapeDtypeStruct((NCORES * NPAD * 8,), _f32),
                     mesh=_MESH, scratch_types=scratch)


def _make_count():
    scratch = [
        pltpu.VMEM((1, CHUNK), jnp.int32),
        pltpu.VMEM((CHUNK,), jnp.int32),
        pltpu.VMEM((CHUNK,), _f32),
        pltpu.VMEM((CRPT,), _f32),
        pltpu.VMEM_SHARED((CPAD,), _f32),
    ]

    def body(dst_hbm, cnt_hbm, idx_d, dstst, ones_v, zc, cacc):
        c = lax.axis_index("c")
        s = lax.axis_index("s")
        wid = c * NTILES + s
        for h in range(CHUNK // 16):
            ones_v[pl.ds(h * 16, 16)] = jnp.ones((16,), _f32)
        for h in range(CRPT // 16):
            zc[pl.ds(h * 16, 16)] = jnp.zeros((16,), _f32)
        pltpu.sync_copy(zc, cacc.at[pl.ds(s * CRPT, CRPT)])
        plsc.subcore_barrier()

        def chunk(j, _):
            pltpu.sync_copy(dst_hbm.at[pl.ds((wid * NCH + j) * CHUNK, CHUNK)],
                            dstst)
            for h in range(CHUNK // 16):
                idx_d[0, pl.ds(h * 16, 16)] = dstst[pl.ds(h * 16, 16)]
            pltpu.sync_copy(ones_v, cacc.at[idx_d.at[0]], add=True)
            return ()
        lax.fori_loop(0, NCH, chunk, ())

        plsc.subcore_barrier()
        pltpu.sync_copy(cacc.at[pl.ds(s * CRPT, CRPT)], zc)
        pltpu.sync_copy(zc, cnt_hbm.at[pl.ds(c * CPAD + s * CRPT, CRPT)])

    return pl.kernel(body, out_type=jax.ShapeDtypeStruct((NCORES * CPAD,), _f32),
                     mesh=_MESH, scratch_types=scratch)


_sc_pass = _make_sc_pass()
_spmm_w8 = _make_spmm_w8()
_count = _make_count()


_BM = 400


@functools.partial(jax.jit, static_argnames=("relu", "nterms", "has_x0"))
def _tc_fused(xs, ws, b, x0, *, relu, nterms, has_x0):
    del nterms

    def body(*refs):
        n = len(xs)
        x_refs = refs[:n]
        w_refs = refs[n:2 * n]
        b_ref = refs[2 * n]
        pos = 2 * n + 1
        if has_x0:
            x0_ref = refs[pos]
            pos += 1
        out_ref = refs[pos]
        acc = jnp.zeros_like(out_ref)
        for xr, wr in zip(x_refs, w_refs):
            acc = acc + jnp.dot(xr[...], wr[...], preferred_element_type=_f32)
        acc = acc + b_ref[...]
        if has_x0:
            acc = acc + x0_ref[...]
        if relu:
            acc = jnp.maximum(acc, 0.0)
        out_ref[...] = acc

    in_specs = []
    for x in xs:
        k = x.shape[1]
        in_specs.append(pl.BlockSpec((_BM, k), lambda i: (i, 0)))
    for w in ws:
        k = w.shape[0]
        in_specs.append(pl.BlockSpec((k, 128), lambda i: (0, 0)))
    in_specs.append(pl.BlockSpec((1, 128), lambda i: (0, 0)))
    args = list(xs) + list(ws) + [b.reshape(1, 128)]
    if has_x0:
        in_specs.append(pl.BlockSpec((_BM, 128), lambda i: (i, 0)))
        args.append(x0)

    return pl.pallas_call(
        body,
        grid=(N // _BM,),
        in_specs=in_specs,
        out_specs=pl.BlockSpec((_BM, 128), lambda i: (i, 0)),
        out_shape=jax.ShapeDtypeStruct((N, 128), _f32),
    )(*args)


def _fused(xs_ws, b, x0=None, relu=True):
    xs = tuple(x for x, _ in xs_ws)
    ws = tuple(w for _, w in xs_ws)
    return _tc_fused(xs, ws, b, x0 if x0 is not None else jnp.zeros((1, 1), _f32),
                     relu=relu, nterms=len(xs), has_x0=x0 is not None)


def _pad_edges(row, col):
    ne = EPAD - row.shape[0]
    ar = jnp.arange(ne, dtype=col.dtype)
    row = jnp.concatenate([row, jnp.zeros((ne,), row.dtype)])
    col = jnp.concatenate([col, N + (ar % 64)])
    return row.astype(jnp.int32), col.astype(jnp.int32)


def _sum_parts(p, d):
    p = p.reshape(NCORES, -1, d) if d > 1 else p.reshape(NCORES, -1)
    return (p[0] + p[1])[:N]


def _r64(a):
    return a.reshape(-1, GCH)


def _r128(a):
    return a.reshape(-1, CHUNK)


def _idx3(srcflat):
    s = _r64(srcflat)
    return jnp.stack([s, s, s], axis=1).reshape(-1)


def _spmm(idx3, dst2d, t3, aux):
    zea, zwe, zflag = aux
    s, c = _sc_pass(idx3, dst2d, zea, zea, t3, zwe, zflag)
    return _sum_parts(s, 128), _sum_parts(c, 1)


def _dep(x, t):
    x, _ = lax.optimization_barrier((x, t[0, 0]))
    return x


def _expand_words(idx2d):
    w = idx2d[:, None, :] * 8 + jnp.arange(8, dtype=jnp.int32)[None, :, None]
    return w.reshape(-1, CHUNK)


def _spmm8(srcw, dstw, x):
    p = _spmm_w8(srcw, dstw, x.reshape(-1)).reshape(NCORES, NPAD, 8)
    return (p[0] + p[1])[:N]


def kernel(game_x, state_x, edge_index_v_v, edge_type_v_v, edge_index_history_v_s, edge_attr_history_v_s, edge_index_in_v_s, edge_index_s_s, W10, b10, Wrel1, Wroot1, b1, Wk3, bk3, Wq3, bq3, Wv3, bv3, We3, be3, Wskip3, b3, Wl32, bl32, Wr32, Wl4, bl4, Wr4, Wl42, bl42, Wr42, W2, b2, Wl5, bl5, Wr5, Wlin, blin):
    zea = jnp.zeros((EPAD,), _f32)
    zwe = jnp.zeros((3 * 128,), _f32)
    zflag = jnp.zeros((16,), jnp.int32)
    aux = (zea, zwe, zflag)
    zpad2n = jnp.zeros((2 * N, 128), _f32)
    vv_s, vv_d = _pad_edges(edge_index_v_v[0], edge_index_v_v[1])
    h_s, h_d = _pad_edges(edge_index_history_v_s[0], edge_index_history_v_s[1])
    in_s, in_d = _pad_edges(edge_index_in_v_s[0], edge_index_in_v_s[1])
    ss_s, ss_d = _pad_edges(edge_index_s_s[0], edge_index_s_s[1])

    vv_i3 = _idx3(vv_s)
    deg = _sum_parts(_count(vv_d), 1)
    dis = jnp.where(deg > 0, lax.rsqrt(jnp.maximum(deg, 1e-20)), 0.0)[:, None]
    vv_sw = _expand_words(_r128(vv_s))
    vv_dw = _expand_words(_r128(vv_d))
    x0 = jnp.pad(game_x, ((0, 0), (0, 1)))
    hs = [x0]
    h = x0
    for _ in range(3):
        s = _spmm8(vv_sw, vv_dw, dis * h)
        h = dis * s
        hs.append(h)
    X4 = jnp.concatenate(hs, axis=1)
    W10r = jnp.concatenate(
        [jnp.pad(W10[i], ((0, 1), (0, 0))) for i in range(4)], axis=0)
    gx = _fused([(X4, W10r)], b10, relu=True)

    et_pad = jnp.concatenate(
        [edge_type_v_v, jnp.full((EPAD - E,), -1, edge_type_v_v.dtype)])
    ar = (jnp.arange(EPAD, dtype=jnp.int32) % 64) + N
    dflat = vv_d
    gx3 = jnp.concatenate([gx, zpad2n], axis=0)
    terms = []
    for r in range(3):
        colr = jnp.where(et_pad == r, dflat, ar)
        s, c = _spmm(vv_i3, colr, gx3, aux)
        terms.append((s / jnp.clip(c, 1.0)[:, None], Wrel1[r]))
    terms.append((gx, Wroot1))
    gx = _fused(terms, b1, relu=True)

    k_ = _fused([(jnp.pad(state_x, ((0, 0), (0, 1))),
                  jnp.pad(Wk3, ((0, 1), (0, 0))))], bk3, relu=False)
    q_ = _fused([(gx, Wq3)], bq3, relu=False)
    v_ = _fused([(gx, Wv3)], bv3, relu=False)
    h_dg = jnp.minimum(h_d, N - 1)
    ea_pad = jnp.concatenate(
        [edge_attr_history_v_s, jnp.zeros((EPAD - E, 2), _f32)])
    ea0 = ea_pad[:, 0]
    ea1 = ea_pad[:, 1]
    we = jnp.concatenate([We3[0], We3[1], be3])
    tqvk = jnp.concatenate([q_, v_, k_], axis=0)
    hr = _r64(h_s)
    src3 = jnp.stack([hr, N + hr, 2 * N + _r64(h_dg)], axis=1).reshape(-1)
    gflag = jnp.ones((16,), jnp.int32)
    p, _hc = _sc_pass(src3, h_d, ea0, ea1, tqvk, we, gflag)
    agg = _sum_parts(p, 128)
    sx = _fused([(jnp.pad(state_x, ((0, 0), (0, 1))),
                  jnp.pad(Wskip3, ((0, 1), (0, 0))))], b3, x0=agg, relu=True)

    gx3 = jnp.concatenate([gx, zpad2n], axis=0)
    h_i3 = _idx3(h_s)
    s, c = _spmm(h_i3, h_d, gx3, aux)
    m = s / jnp.clip(c, 1.0)[:, None]
    sx = _fused([(m, Wl32), (sx, Wr32)], bl32, relu=True)

    s, c = _spmm(_idx3(in_s), in_d, gx3, aux)
    m = s / jnp.clip(c, 1.0)[:, None]
    sx = _fused([(m, Wl4), (sx, Wr4)], bl4, relu=True)
    sx = _fused([(m, Wl42), (sx, Wr42)], bl42, relu=True)

    ss_i3 = _idx3(ss_s)
    deg = _sum_parts(_count(ss_d), 1)
    dis = jnp.where(deg > 0, lax.rsqrt(jnp.maximum(deg, 1e-20)), 0.0)[:, None]
    hs = [sx]
    h = sx
    for _ in range(3):
        s, _ = _spmm(ss_i3, ss_d,
                     jnp.concatenate([dis * h, zpad2n], axis=0), aux)
        h = dis * s
        hs.append(h)
    sx = _fused([(hs[i], W2[i]) for i in range(4)], b2, relu=True)

    s, c = _spmm(ss_i3, ss_d, jnp.concatenate([sx, zpad2n], axis=0), aux)
    m = s / jnp.clip(c, 1.0)[:, None]
    sx = _fused([(m, Wl5), (sx, Wr5)], bl5, relu=True)

    Wlin_pad = jnp.pad(Wlin, ((0, 0), (0, 120)))
    blin_pad = jnp.pad(blin, (0, 120))
    out = _fused([(sx, Wlin_pad)], blin_pad, relu=False)
    return out[:, :8]


# speedup vs baseline: 3.6737x; 1.0815x over previous
import functools

import jax
import jax.numpy as jnp
from jax import lax
from jax.experimental import pallas as pl
from jax.experimental.pallas import tpu as pltpu
from jax.experimental.pallas import tpu_sc as plsc

N = 10000
NPAD = 10112
CPAD = 10240
E = 320000
CHUNK = 128
NCH = 80
NTILES = 16
NCORES = 2
EPAD = NCORES * NTILES * NCH * CHUNK
RPT = NPAD // NTILES
CRPT = CPAD // NTILES

_f32 = jnp.float32
_MESH = plsc.VectorSubcoreMesh(core_axis_name="c", subcore_axis_name="s")


def _zero_vmem(ref, rows, d):
    for i in range(rows):
        for h in range(d // 16):
            ref[i, pl.ds(h * 16, 16)] = jnp.zeros((16,), _f32)


GCH = 64
GNCH = EPAD // (NCORES * NTILES * GCH)


def _make_sc_pass():
    d = 128
    scratch = [
        pltpu.VMEM((8 * 3 * GCH,), jnp.int32),
        pltpu.VMEM((8 * GCH,), jnp.int32),
        pltpu.VMEM((1, GCH), jnp.int32),
        pltpu.VMEM((GCH, d), _f32),
        pltpu.VMEM((GCH, d), _f32),
        pltpu.VMEM((GCH, d), _f32),
        pltpu.VMEM((GCH, d), _f32),
        pltpu.VMEM((GCH + 16,), _f32),
        pltpu.VMEM((GCH + 16,), _f32),
        pltpu.VMEM((3 * d,), _f32),
        pltpu.VMEM((16,), jnp.int32),
        pltpu.VMEM((8, d), _f32),
        pltpu.VMEM((GCH,), _f32),
        pltpu.VMEM((CRPT,), _f32),
        pltpu.VMEM_SHARED((NPAD, d), _f32),
        pltpu.VMEM_SHARED((CPAD,), _f32),
        pltpu.SemaphoreType.DMA,
    ]

    def body(idx3_hbm, dst_hbm, ea0_hbm, ea1_hbm, t_hbm, we_hbm, flag_hbm,
             out_hbm, cnt_hbm, idx3, dstst, idx_d, buf0a, buf0b, buf1, buf2,
             ea0_v, ea1_v, we_v, flag_v, zbuf, ones_v, zc, acc, cacc, sem):
        c = lax.axis_index("c")
        s = lax.axis_index("s")
        wid = c * NTILES + s

        _zero_vmem(zbuf, 8, d)
        for h in range(GCH // 16):
            ones_v[pl.ds(h * 16, 16)] = jnp.ones((16,), _f32)
        for h in range(CRPT // 16):
            zc[pl.ds(h * 16, 16)] = jnp.zeros((16,), _f32)

        base = s * RPT

        def zero_acc(i, _):
            pltpu.sync_copy(zbuf, acc.at[pl.ds(base + i * 8, 8)])
            return ()
        lax.fori_loop(0, RPT // 8, zero_acc, ())
        pltpu.sync_copy(zc, cacc.at[pl.ds(s * CRPT, CRPT)])

        pltpu.sync_copy(we_hbm, we_v)
        pltpu.sync_copy(flag_hbm, flag_v)
        gated = flag_v[pl.ds(0, 16)][0] == 1
        plsc.subcore_barrier()

        def load_blk(bj):
            pltpu.sync_copy(
                idx3_hbm.at[pl.ds((wid * GNCH + bj) * 3 * GCH, 8 * 3 * GCH)],
                idx3)
            pltpu.sync_copy(
                dst_hbm.at[pl.ds((wid * GNCH + bj) * GCH, 8 * GCH)], dstst)

        def fire(j, buf):
            return pltpu.async_copy(
                t_hbm.at[idx3.at[pl.ds((j % 8) * 3 * GCH, GCH)]], buf, sem)

        load_blk(0)
        fire(0, buf0a)

        def process(j, cur, nxt):
            boff = (j % 8) * GCH
            for h in range(GCH // 16):
                idx_d[0, pl.ds(h * 16, 16)] = dstst[pl.ds(boff + h * 16, 16)]
            pltpu.make_async_copy(t_hbm.at[pl.ds(0, GCH)], cur, sem).wait()

            @pl.when(gated)
            def _():
                pltpu.async_copy(
                    t_hbm.at[idx3.at[pl.ds((j % 8) * 3 * GCH + GCH, GCH)]],
                    buf1, sem).wait()
                pltpu.async_copy(
                    t_hbm.at[idx3.at[pl.ds((j % 8) * 3 * GCH + 2 * GCH, GCH)]],
                    buf2, sem).wait()
                gbase = (wid * GNCH + j) * GCH
                pltpu.sync_copy(ea0_hbm.at[pl.ds(gbase, GCH)],
                                ea0_v.at[pl.ds(0, GCH)])
                pltpu.sync_copy(ea1_hbm.at[pl.ds(gbase, GCH)],
                                ea1_v.at[pl.ds(0, GCH)])

                def edge(i, _):
                    a0 = ea0_v[pl.ds(i, 16)][0]
                    a1 = ea1_v[pl.ds(i, 16)][0]
                    for h in range(d // 16):
                        sl = pl.ds(h * 16, 16)
                        t = (buf2[i, sl] + cur[i, sl]
                             + a0 * we_v[pl.ds(h * 16, 16)]
                             + a1 * we_v[pl.ds(d + h * 16, 16)]
                             + we_v[pl.ds(2 * d + h * 16, 16)])
                        eta = 1.0 / (1.0 + jnp.exp(-t))
                        cur[i, sl] = eta * buf1[i, sl]
                    return ()
                lax.fori_loop(0, GCH, edge, ())

            @pl.when(j + 1 < GNCH)
            def _():
                @pl.when((j + 1) % 8 == 0)
                def _():
                    load_blk(j + 1)
                fire(j + 1, nxt)

            pltpu.sync_copy(cur, acc.at[idx_d.at[0]], add=True)
            pltpu.sync_copy(ones_v, cacc.at[idx_d.at[0]], add=True)

        def pair(i, _):
            process(2 * i, buf0a, buf0b)
            process(2 * i + 1, buf0b, buf0a)
            return ()
        lax.fori_loop(0, GNCH // 2, pair, ())

        plsc.subcore_barrier()

        def wb(i, _):
            pltpu.sync_copy(acc.at[pl.ds(base + i * 8, 8)], zbuf)
            pltpu.sync_copy(zbuf, out_hbm.at[pl.ds(c * NPAD + base + i * 8, 8)])
            return ()
        lax.fori_loop(0, RPT // 8, wb, ())
        pltpu.sync_copy(cacc.at[pl.ds(s * CRPT, CRPT)], zc)
        pltpu.sync_copy(zc, cnt_hbm.at[pl.ds(c * CPAD + s * CRPT, CRPT)])

    return pl.kernel(
        body,
        out_type=(jax.ShapeDtypeStruct((NCORES * NPAD, d), _f32),
                  jax.ShapeDtypeStruct((NCORES * CPAD,), _f32)),
        mesh=_MESH, scratch_types=scratch)


def _make_spmm_w8():
    D8 = 8
    ZB = 1264
    scratch = [
        pltpu.VMEM((D8, CHUNK), jnp.int32),
        pltpu.VMEM((D8, CHUNK), jnp.int32),
        pltpu.VMEM((D8, CHUNK), _f32),
        pltpu.VMEM((ZB,), _f32),
        pltpu.VMEM_SHARED((NPAD * D8,), _f32),
        pltpu.SemaphoreType.DMA,
    ]

    def body(srcw_hbm, dstw_hbm, x_hbm, out_hbm, idx_s, idx_d, rows, zbuf,
             acc, sem):
        c = lax.axis_index("c")
        s = lax.axis_index("s")
        wid = c * NTILES + s
        for h in range(ZB // 16):
            zbuf[pl.ds(h * 16, 16)] = jnp.zeros((16,), _f32)
        base = s * RPT * D8

        def za(i, _):
            pltpu.sync_copy(zbuf, acc.at[pl.ds(base + i * ZB, ZB)])
            return ()
        lax.fori_loop(0, 4, za, ())
        plsc.subcore_barrier()

        def chunk(j, _):
            pltpu.sync_copy(srcw_hbm.at[pl.ds((wid * NCH + j) * D8, D8)], idx_s)
            pltpu.sync_copy(dstw_hbm.at[pl.ds((wid * NCH + j) * D8, D8)], idx_d)
            descs = [pltpu.async_copy(x_hbm.at[idx_s.at[k]], rows.at[k], sem)
                     for k in range(D8)]
            for desc in descs:
                desc.wait()
            for k in range(D8):
                pltpu.sync_copy(rows.at[k], acc.at[idx_d.at[k]], add=True)
            return ()
        lax.fori_loop(0, NCH, chunk, ())

        plsc.subcore_barrier()

        def wbw(i, _):
            pltpu.sync_copy(acc.at[pl.ds(base + i * ZB, ZB)], zbuf)
            pltpu.sync_copy(
                zbuf, out_hbm.at[pl.ds(c * NPAD * D8 + base + i * ZB, ZB)])
            return ()
        lax.fori_loop(0, 4, wbw, ())

    return pl.kernel(body,
                     out_type=jax.ShapeDtypeStruct((NCORES * NPAD * 8,), _f32),
                     mesh=_MESH, scratch_types=scratch)


def _make_count():
    scratch = [
        pltpu.VMEM((1, CHUNK), jnp.int32),
        pltpu.VMEM((CHUNK,), jnp.int32),
        pltpu.VMEM((CHUNK,), _f32),
        pltpu.VMEM((CRPT,), _f32),
        pltpu.VMEM_SHARED((CPAD,), _f32),
    ]

    def body(dst_hbm, cnt_hbm, idx_d, dstst, ones_v, zc, cacc):
        c = lax.axis_index("c")
        s = lax.axis_index("s")
        wid = c * NTILES + s
        for h in range(CHUNK // 16):
            ones_v[pl.ds(h * 16, 16)] = jnp.ones((16,), _f32)
        for h in range(CRPT // 16):
            zc[pl.ds(h * 16, 16)] = jnp.zeros((16,), _f32)
        pltpu.sync_copy(zc, cacc.at[pl.ds(s * CRPT, CRPT)])
        plsc.subcore_barrier()

        def chunk(j, _):
            pltpu.sync_copy(dst_hbm.at[pl.ds((wid * NCH + j) * CHUNK, CHUNK)],
                            dstst)
            for h in range(CHUNK // 16):
                idx_d[0, pl.ds(h * 16, 16)] = dstst[pl.ds(h * 16, 16)]
            pltpu.sync_copy(ones_v, cacc.at[idx_d.at[0]], add=True)
            return ()
        lax.fori_loop(0, NCH, chunk, ())

        plsc.subcore_barrier()
        pltpu.sync_copy(cacc.at[pl.ds(s * CRPT, CRPT)], zc)
        pltpu.sync_copy(zc, cnt_hbm.at[pl.ds(c * CPAD + s * CRPT, CRPT)])

    return pl.kernel(body, out_type=jax.ShapeDtypeStruct((NCORES * CPAD,), _f32),
                     mesh=_MESH, scratch_types=scratch)


_sc_pass = _make_sc_pass()
_spmm_w8 = _make_spmm_w8()
_count = _make_count()


_BM = 400


@functools.partial(jax.jit, static_argnames=("relu", "nterms", "has_x0"))
def _tc_fused(xs, ws, b, x0, *, relu, nterms, has_x0):
    del nterms

    def body(*refs):
        n = len(xs)
        x_refs = refs[:n]
        w_refs = refs[n:2 * n]
        b_ref = refs[2 * n]
        pos = 2 * n + 1
        if has_x0:
            x0_ref = refs[pos]
            pos += 1
        out_ref = refs[pos]
        acc = jnp.zeros_like(out_ref)
        for xr, wr in zip(x_refs, w_refs):
            acc = acc + jnp.dot(xr[...], wr[...], preferred_element_type=_f32)
        acc = acc + b_ref[...]
        if has_x0:
            acc = acc + x0_ref[...]
        if relu:
            acc = jnp.maximum(acc, 0.0)
        out_ref[...] = acc

    in_specs = []
    for x in xs:
        k = x.shape[1]
        in_specs.append(pl.BlockSpec((_BM, k), lambda i: (i, 0)))
    for w in ws:
        k = w.shape[0]
        in_specs.append(pl.BlockSpec((k, 128), lambda i: (0, 0)))
    in_specs.append(pl.BlockSpec((1, 128), lambda i: (0, 0)))
    args = list(xs) + list(ws) + [b.reshape(1, 128)]
    if has_x0:
        in_specs.append(pl.BlockSpec((_BM, 128), lambda i: (i, 0)))
        args.append(x0)

    return pl.pallas_call(
        body,
        grid=(N // _BM,),
        in_specs=in_specs,
        out_specs=pl.BlockSpec((_BM, 128), lambda i: (i, 0)),
        out_shape=jax.ShapeDtypeStruct((N, 128), _f32),
    )(*args)


def _fused(xs_ws, b, x0=None, relu=True):
    xs = tuple(x for x, _ in xs_ws)
    ws = tuple(w for _, w in xs_ws)
    return _tc_fused(xs, ws, b, x0 if x0 is not None else jnp.zeros((1, 1), _f32),
                     relu=relu, nterms=len(xs), has_x0=x0 is not None)


def _pad_edges(row, col):
    ne = EPAD - row.shape[0]
    ar = jnp.arange(ne, dtype=col.dtype)
    row = jnp.concatenate([row, jnp.zeros((ne,), row.dtype)])
    col = jnp.concatenate([col, N + (ar % 64)])
    return row.astype(jnp.int32), col.astype(jnp.int32)


def _sum_parts(p, d):
    p = p.reshape(NCORES, -1, d) if d > 1 else p.reshape(NCORES, -1)
    return (p[0] + p[1])[:N]


def _r64(a):
    return a.reshape(-1, GCH)


def _r128(a):
    return a.reshape(-1, CHUNK)


def _idx3(srcflat):
    s = _r64(srcflat)
    return jnp.stack([s, s, s], axis=1).reshape(-1)


def _spmm(idx3, dst2d, t3, aux):
    zea, zwe, zflag = aux
    s, c = _sc_pass(idx3, dst2d, zea, zea, t3, zwe, zflag)
    return _sum_parts(s, 128), _sum_parts(c, 1)


def _dep(x, t):
    x, _ = lax.optimization_barrier((x, t[0, 0]))
    return x


def _expand_words(idx2d):
    w = idx2d[:, None, :] * 8 + jnp.arange(8, dtype=jnp.int32)[None, :, None]
    return w.reshape(-1, CHUNK)


def _spmm8(srcw, dstw, x):
    p = _spmm_w8(srcw, dstw, x.reshape(-1)).reshape(NCORES, NPAD, 8)
    return (p[0] + p[1])[:N]


def kernel(game_x, state_x, edge_index_v_v, edge_type_v_v, edge_index_history_v_s, edge_attr_history_v_s, edge_index_in_v_s, edge_index_s_s, W10, b10, Wrel1, Wroot1, b1, Wk3, bk3, Wq3, bq3, Wv3, bv3, We3, be3, Wskip3, b3, Wl32, bl32, Wr32, Wl4, bl4, Wr4, Wl42, bl42, Wr42, W2, b2, Wl5, bl5, Wr5, Wlin, blin):
    zea = jnp.zeros((EPAD,), _f32)
    zwe = jnp.zeros((3 * 128,), _f32)
    zflag = jnp.zeros((16,), jnp.int32)
    aux = (zea, zwe, zflag)
    zpad2n = jnp.zeros((2 * N, 128), _f32)
    vv_s, vv_d = _pad_edges(edge_index_v_v[0], edge_index_v_v[1])
    h_s, h_d = _pad_edges(edge_index_history_v_s[0], edge_index_history_v_s[1])
    in_s, in_d = _pad_edges(edge_index_in_v_s[0], edge_index_in_v_s[1])
    ss_s, ss_d = _pad_edges(edge_index_s_s[0], edge_index_s_s[1])

    vv_i3 = _idx3(vv_s)
    deg = _sum_parts(_count(vv_d), 1)
    dis = jnp.where(deg > 0, lax.rsqrt(jnp.maximum(deg, 1e-20)), 0.0)[:, None]
    vv_sw = _expand_words(_r128(vv_s))
    vv_dw = _expand_words(_r128(vv_d))
    x0 = jnp.pad(game_x, ((0, 0), (0, 1)))
    hs = [x0]
    h = x0
    for _ in range(3):
        s = _spmm8(vv_sw, vv_dw, dis * h)
        h = dis * s
        hs.append(h)
    X4 = jnp.concatenate(hs, axis=1)
    W10r = jnp.concatenate(
        [jnp.pad(W10[i], ((0, 1), (0, 0))) for i in range(4)], axis=0)
    gx = _fused([(X4, W10r)], b10, relu=True)

    et_pad = jnp.concatenate(
        [edge_type_v_v, jnp.full((EPAD - E,), -1, edge_type_v_v.dtype)])
    ar = (jnp.arange(EPAD, dtype=jnp.int32) % 64) + N
    dflat = vv_d
    gx3 = jnp.concatenate([gx, zpad2n], axis=0)
    terms = []
    for r in range(3):
        colr = jnp.where(et_pad == r, dflat, ar)
        s, c = _spmm(vv_i3, colr, gx3, aux)
        terms.append((s / jnp.clip(c, 1.0)[:, None], Wrel1[r]))
    terms.append((gx, Wroot1))
    gx = _fused(terms, b1, relu=True)

    k_ = _fused([(jnp.pad(state_x, ((0, 0), (0, 1))),
                  jnp.pad(Wk3, ((0, 1), (0, 0))))], bk3, relu=False)
    q_ = _fused([(gx, Wq3)], bq3, relu=False)
    v_ = _fused([(gx, Wv3)], bv3, relu=False)
    h_dg = jnp.minimum(h_d, N - 1)
    ea_pad = jnp.concatenate(
        [edge_attr_history_v_s, jnp.zeros((EPAD - E, 2), _f32)])
    ea0 = ea_pad[:, 0]
    ea1 = ea_pad[:, 1]
    we = jnp.concatenate([We3[0], We3[1], be3])
    tqvk = jnp.concatenate([q_, v_, k_], axis=0)
    hr = _r64(h_s)
    src3 = jnp.stack([hr, N + hr, 2 * N + _r64(h_dg)], axis=1).reshape(-1)
    gflag = jnp.ones((16,), jnp.int32)
    p, _hc = _sc_pass(src3, h_d, ea0, ea1, tqvk, we, gflag)
    agg = _sum_parts(p, 128)
    sx = _fused([(jnp.pad(state_x, ((0, 0), (0, 1))),
                  jnp.pad(Wskip3, ((0, 1), (0, 0))))], b3, x0=agg, relu=True)

    gx3 = jnp.concatenate([gx, zpad2n], axis=0)
    h_i3 = _idx3(h_s)
    s, c = _spmm(h_i3, h_d, gx3, aux)
    m = s / jnp.clip(c, 1.0)[:, None]
    sx = _fused([(m, Wl32), (sx, Wr32)], bl32, relu=True)

    s, c = _spmm(_idx3(in_s), in_d, gx3, aux)
    m = s / jnp.clip(c, 1.0)[:, None]
    sx = _fused([(m, Wl4), (sx, Wr4)], bl4, relu=True)
    sx = _fused([(m, Wl42), (sx, Wr42)], bl42, relu=True)

    ss_i3 = _idx3(ss_s)
    deg = _sum_parts(_count(ss_d), 1)
    dis = jnp.where(deg > 0, lax.rsqrt(jnp.maximum(deg, 1e-20)), 0.0)[:, None]
    hs = [sx]
    h = sx
    for _ in range(3):
        s, _ = _spmm(ss_i3, ss_d,
                     jnp.concatenate([dis * h, zpad2n], axis=0), aux)
        h = dis * s
        hs.append(h)
    sx = _fused([(hs[i], W2[i]) for i in range(4)], b2, relu=True)

    s, c = _spmm(ss_i3, ss_d, jnp.concatenate([sx, zpad2n], axis=0), aux)
    m = s / jnp.clip(c, 1.0)[:, None]
    sx = _fused([(m, Wl5), (sx, Wr5)], bl5, relu=True)

    Wlin_pad = jnp.pad(Wlin, ((0, 0), (0, 120)))
    blin_pad = jnp.pad(blin, (0, 120))
    out = _fused([(sx, Wlin_pad)], blin_pad, relu=False)
    return out[:, :8]
